# baseline (speedup 1.0000x reference)
import functools

import jax
import jax.numpy as jnp
import numpy as np
from jax import lax
from jax.experimental import pallas as pl
from jax.experimental.pallas import tpu as pltpu
from jax.experimental.pallas import tpu_sc as plsc

EPS = 1e-5

_NC = 2
_NS = 16
_LG = 128
_GB = 4


def _sel_gate(n_chan):
    c = lax.broadcasted_iota(jnp.int32, (n_chan, 3 * n_chan), 0)
    k = lax.broadcasted_iota(jnp.int32, (n_chan, 3 * n_chan), 1)
    return (k // 3 == c).astype(jnp.float32)


def _ln_s(s):
    mu = jnp.mean(s, axis=1, keepdims=True)
    var = jnp.mean((s - mu) * (s - mu), axis=1, keepdims=True)
    return (s - mu) * lax.rsqrt(var + EPS)


def _ln_v(v, n_chan):
    m = jnp.sum(v * v, axis=1, keepdims=True) * (1.0 / n_chan)
    return v * lax.rsqrt(m + EPS)


def _silu(x):
    return x * jax.nn.sigmoid(x)


def _dgt(xt, w):
    return lax.dot_general(xt, w, (((0,), (0,)), ((), ())),
                           preferred_element_type=jnp.float32)


def _ln_t(xt):
    mu = jnp.mean(xt, axis=0, keepdims=True)
    var = jnp.mean((xt - mu) * (xt - mu), axis=0, keepdims=True)
    return (xt - mu) * lax.rsqrt(var + EPS)


def _edge_body(es_ref, ev0_ref, ev1_ref, ev2_ref,
               wesw, wesb, wevw, wevb,
               esw01, esb01, evw01, evb01,
               s01, vch):
    est = es_ref[...]
    evt = jnp.concatenate([ev0_ref[...], ev1_ref[...], ev2_ref[...]], axis=0)
    esn_t = _ln_t(est)
    m = jnp.sum(evt * evt, axis=0, keepdims=True)
    evn_t = evt * lax.rsqrt(m + EPS)
    hes = _dgt(esn_t, wesw[...]) + wesb[...]
    vraw = _dgt(evn_t, wevw[...]) + wevb[...]
    pre = hes @ esw01[...] + esb01[...]
    ec = pre * jax.nn.sigmoid(pre)
    gc = jnp.concatenate([ec[:, 0:4], ec[:, 64:68], hes[:, :1]], axis=1)
    sg = jax.nn.sigmoid(gc)
    sel8 = _sel_gate(8)
    g8 = sg[:, 0:8]
    gg = (g8 * sg[:, 8:9]) @ sel8
    g12 = g8 @ sel8
    t01 = vraw @ evw01[...]
    ecv01 = t01 * gg + evb01[...] * g12
    s01[...] = ec
    b = ec.shape[0]
    vch[...] = jnp.concatenate(
        [ecv01, jnp.ones((b, 1), jnp.float32),
         jnp.zeros((b, 7), jnp.float32)], axis=1)


def _edge_call(edge_s_t, edge_v_comps, we, l0, l1, e_pad_h, block_e, blk0,
               last_real):
    n_blocks = e_pad_h // block_e

    def col_map(i):
        return (0, jnp.minimum(blk0 + i, last_real))

    def full(w):
        return pl.BlockSpec(w.shape, lambda i: (0,) * w.ndim)

    r = lambda x: jnp.reshape(x, (1, -1))
    esw01 = jnp.concatenate([l0['sw'], l1['sw']], axis=1)
    esb01 = jnp.concatenate([r(l0['sb']), r(l1['sb'])], axis=1)
    evw01 = jnp.concatenate([l0['vw'], l1['vw']], axis=1)
    evb01 = jnp.concatenate([r(l0['vb']), r(l1['vb'])], axis=1)
    weights = [we['sw'], r(we['sb']), we['vw'], r(we['vb']),
               esw01, esb01, evw01, evb01]
    out_shape = [jax.ShapeDtypeStruct((e_pad_h, 128), jnp.float32),
                 jax.ShapeDtypeStruct((e_pad_h, 32), jnp.float32)]
    return pl.pallas_call(
        _edge_body,
        grid=(n_blocks,),
        in_specs=[pl.BlockSpec((32, block_e), col_map)] +
                 [pl.BlockSpec((1, block_e), col_map)] * 3 +
                 [full(w) for w in weights],
        out_specs=[pl.BlockSpec((block_e, 128), lambda i: (i, 0)),
                   pl.BlockSpec((block_e, 32), lambda i: (i, 0))],
        out_shape=out_shape,
    )(edge_s_t, *edge_v_comps, *weights)


def _sc_scatter(dst2d, p1, p2, zeros, n_acc, dst_off):
    g = p1.shape[0]
    gpt = g // (_NC * _NS)
    nb = gpt // _GB
    pt = n_acc // _NS
    mesh = plsc.VectorSubcoreMesh(core_axis_name="c", subcore_axis_name="s")

    @functools.partial(
        pl.kernel, mesh=mesh,
        compiler_params=pltpu.CompilerParams(use_tc_tiling_on_sc=False),
        out_type=jax.ShapeDtypeStruct((10, n_acc, 32), jnp.float32),
        scratch_types=[
            pltpu.VMEM_SHARED((n_acc, 32), jnp.float32),
            pltpu.VMEM((gpt, _LG), jnp.int32),
            pltpu.VMEM((_GB, _LG, 32), jnp.float32),
            pltpu.SemaphoreType.DMA,
        ],
    )
    def k(dst_hbm, p1_hbm, p2_hbm, z_hbm, out_hbm, acc, idxs, rowb, sem):
        cid = lax.axis_index("c")
        sid = lax.axis_index("s")
        tid = cid * _NS + sid
        row0 = tid * gpt
        my_rows = pl.ds(sid * pt, pt)
        pltpu.sync_copy(dst_hbm.at[pl.ds(dst_off + row0, gpt)], idxs)
        for ci in range(5):
            pltpu.sync_copy(z_hbm.at[my_rows], acc.at[my_rows])
            plsc.subcore_barrier()

            def body(b, _, ci=ci):
                r = row0 + b * _GB
                if ci < 4:
                    pltpu.sync_copy(
                        p1_hbm.at[pl.ds(r, _GB), :, pl.ds(ci * 32, 32)], rowb)
                else:
                    pltpu.sync_copy(p2_hbm.at[pl.ds(r, _GB)], rowb)
                descs = [
                    pltpu.async_copy(rowb.at[j], acc.at[idxs.at[b * _GB + j]],
                                     sem, add=True)
                    for j in range(_GB)
                ]
                for d in descs:
                    d.wait()
                return _

            lax.fori_loop(0, nb, body, 0)
            plsc.subcore_barrier()
            pltpu.sync_copy(acc.at[my_rows],
                            out_hbm.at[ci * 2 + cid, my_rows])

    return k(dst2d, p1, p2, zeros)


def _node_body(ns_ref, nv_ref, pa_ref, pb_ref, pc_ref, pd_ref,
               wvsw, wvsb, wvvw, wvvb,
               n0sw, n0sb, n0vw, n0vb, r0sw, r0sb, r0vw, r0vb,
               n1sw, n1sb, n1vw, n1vb, r1sw, r1sb, r1vw, r1vb,
               o1sw, o1sb, o2sw, o2sb,
               out_ref):
    sel = _sel_gate(4)
    s_t = _ln_t(ns_ref[...])
    v_t = nv_ref[...]
    m = jnp.sum(v_t * v_t, axis=0, keepdims=True) * 0.25
    vn_t = v_t * lax.rsqrt(m + EPS)
    hs = _dgt(s_t, wvsw[...]) + wvsb[...]
    hv = (_dgt(vn_t, wvvw[...]) + wvvb[...]) * (jax.nn.sigmoid(hs[:, :4]) @ sel)

    p = (pa_ref[...] + pb_ref[...]) + (pc_ref[...] + pd_ref[...])
    pv = p[8] + p[9]
    inv = 1.0 / jnp.maximum(pv[:, 24:25], 1.0)
    aggs = (jnp.concatenate([p[0] + p[1], p[2] + p[3]], axis=1) * inv,
            jnp.concatenate([p[4] + p[5], p[6] + p[7]], axis=1) * inv)
    aggv = (pv[:, 0:12] * inv, pv[:, 12:24] * inv)

    layer_w = ((n0sw, n0sb, n0vw, n0vb, r0sw, r0sb, r0vw, r0vb),
               (n1sw, n1sb, n1vw, n1vb, r1sw, r1sb, r1vw, r1vb))
    for l, (nsw, nsb, nvw, nvb, rsw, rsb, rvw, rvb) in enumerate(layer_w):
        nsn = _ln_s(hs)
        nvn = _ln_v(hv, 4)
        xs = nsn + aggs[l]
        xv = nvn + aggv[l]
        h = _silu(xs @ nsw[...] + nsb[...])
        hv2 = (xv @ nvw[...] + nvb[...]) * (jax.nn.sigmoid(h[:, :4]) @ sel)
        hs = h + nsn @ rsw[...] + rsb[...]
        hv = hv2 + nvn @ rvw[...] + rvb[...]

    nsn = _ln_s(hs)
    h1 = nsn @ o1sw[...] + o1sb[...]
    out_ref[...] = h1 @ o2sw[...] + o2sb[...]


def _node_call(node_s_t, node_v_t, partials, params, block_n):
    n = node_s_t.shape[1]
    n_blocks = -(-n // block_n)

    def full(w):
        return pl.BlockSpec(w.shape, lambda i: (0,) * w.ndim)

    r = lambda x: jnp.reshape(x, (1, -1))
    wv = params['W_v']
    wvvw_perm = jnp.reshape(wv['vw'], (4, 3, 12)).transpose(1, 0, 2).reshape(12, 12)
    lw = []
    for lp in params['layers']:
        ng = lp['node_gvp']
        lw += [ng['sw'], r(ng['sb']), ng['vw'], r(ng['vb']),
               lp['res_s_w'], r(lp['res_s_b']), lp['res_v_w'], r(lp['res_v_b'])]
    o1 = params['W_out1']
    o2 = params['W_out2']
    weights = ([wv['sw'], r(wv['sb']), wvvw_perm, r(wv['vb'])] + lw +
               [o1['sw'], r(o1['sb']), o2['sw'], r(o2['sb'])])
    return pl.pallas_call(
        _node_body,
        grid=(n_blocks,),
        in_specs=[pl.BlockSpec((78, block_n), lambda i: (0, i)),
                  pl.BlockSpec((12, block_n), lambda i: (0, i)),
                  pl.BlockSpec((10, block_n, 32), lambda i: (0, i, 0)),
                  pl.BlockSpec((10, block_n, 32), lambda i: (0, i, 0)),
                  pl.BlockSpec((10, block_n, 32), lambda i: (0, i, 0)),
                  pl.BlockSpec((10, block_n, 32), lambda i: (0, i, 0))] +
                 [full(w) for w in weights],
        out_specs=pl.BlockSpec((block_n, 4), lambda i: (i, 0)),
        out_shape=jax.ShapeDtypeStruct((n, 4), jnp.float32),
    )(node_s_t, node_v_t, *partials, *weights)


def kernel(node_s, node_v, edge_index, edge_s, edge_v, params):
    n = node_s.shape[0]
    e = edge_s.shape[0]
    block_e = 2048
    block_n = 1024
    nq = 4
    q_size = -(-e // (nq * block_e)) * block_e
    edges_per_tile_batch = _NC * _NS * _LG * _GB
    e_pad_q = -(-q_size // edges_per_tile_batch) * edges_per_tile_batch
    n_acc = -(-(n + 1) // (_NS * 8)) * (_NS * 8)

    edge_s_t = edge_s.T
    edge_v_comps = [edge_v[:, 0, c].reshape(1, e) for c in range(3)]
    node_s_t = node_s.T
    node_v_t = jnp.transpose(node_v, (2, 1, 0)).reshape(12, n)
    dst = edge_index[1]
    trash = n_acc - 1
    drows = []
    for q in range(nq):
        s = q * q_size
        l = min(q_size, e - s)
        drows.append(jnp.concatenate(
            [dst[s:s + l], jnp.full((e_pad_q - l,), trash, jnp.int32)]))
    dst2d = jnp.reshape(jnp.concatenate(drows), (nq * e_pad_q // _LG, _LG))
    zeros = jnp.zeros((n_acc, 32), jnp.float32)

    last_real = -(-e // block_e) - 1
    partials = []
    for q in range(nq):
        blk0 = (q * q_size) // block_e
        p1, p2 = _edge_call(edge_s_t, edge_v_comps, params['W_e'],
                            params['layers'][0]['edge_gvp'],
                            params['layers'][1]['edge_gvp'], e_pad_q, block_e,
                            blk0, last_real)
        p1r = jnp.reshape(p1, (e_pad_q // _LG, _LG, 128))
        p2r = jnp.reshape(p2, (e_pad_q // _LG, _LG, 32))
        partials.append(_sc_scatter(dst2d, p1r, p2r, zeros, n_acc,
                                    q * (e_pad_q // _LG)))
    return _node_call(node_s_t, node_v_t, partials, params, block_n)


# speedup vs baseline: 1.2567x; 1.2567x over previous
import functools

import jax
import jax.numpy as jnp
import numpy as np
from jax import lax
from jax.experimental import pallas as pl
from jax.experimental.pallas import tpu as pltpu
from jax.experimental.pallas import tpu_sc as plsc

EPS = 1e-5

_NC = 2
_NS = 16
_LG = 128
_GB = 4


def _sel_gate(n_chan):
    c = lax.broadcasted_iota(jnp.int32, (n_chan, 3 * n_chan), 0)
    k = lax.broadcasted_iota(jnp.int32, (n_chan, 3 * n_chan), 1)
    return (k // 3 == c).astype(jnp.float32)


def _ln_s(s):
    mu = jnp.mean(s, axis=1, keepdims=True)
    var = jnp.mean((s - mu) * (s - mu), axis=1, keepdims=True)
    return (s - mu) * lax.rsqrt(var + EPS)


def _ln_v(v, n_chan):
    m = jnp.sum(v * v, axis=1, keepdims=True) * (1.0 / n_chan)
    return v * lax.rsqrt(m + EPS)


def _silu(x):
    return x * jax.nn.sigmoid(x)


def _dgt(xt, w):
    return lax.dot_general(xt, w, (((0,), (0,)), ((), ())),
                           preferred_element_type=jnp.float32)


def _ln_t(xt):
    mu = jnp.mean(xt, axis=0, keepdims=True)
    var = jnp.mean((xt - mu) * (xt - mu), axis=0, keepdims=True)
    return (xt - mu) * lax.rsqrt(var + EPS)


def _edge_body(es_ref, ev0_ref, ev1_ref, ev2_ref,
               wesw, wesb, wevw, wevb,
               esw01, esb01, evw01, evb01,
               s01, vch):
    est = es_ref[...]
    evt = jnp.concatenate([ev0_ref[...], ev1_ref[...], ev2_ref[...]], axis=0)
    esn_t = _ln_t(est)
    m = jnp.sum(evt * evt, axis=0, keepdims=True)
    evn_t = evt * lax.rsqrt(m + EPS)
    hes = _dgt(esn_t, wesw[...]) + wesb[...]
    vraw = _dgt(evn_t, wevw[...]) + wevb[...]
    pre = hes @ esw01[...] + esb01[...]
    ec = pre * jax.nn.sigmoid(pre)
    gc = jnp.concatenate([ec[:, 0:4], ec[:, 64:68], hes[:, :1]], axis=1)
    sg = jax.nn.sigmoid(gc)
    sel8 = _sel_gate(8)
    g8 = sg[:, 0:8]
    gg = (g8 * sg[:, 8:9]) @ sel8
    g12 = g8 @ sel8
    t01 = vraw @ evw01[...]
    ecv01 = t01 * gg + evb01[...] * g12
    s01[...] = ec
    b = ec.shape[0]
    vch[...] = jnp.concatenate(
        [ecv01, jnp.ones((b, 1), jnp.float32),
         jnp.zeros((b, 7), jnp.float32)], axis=1)


def _edge_call(edge_s_t, edge_v_comps, we, l0, l1, e_pad_h, block_e, blk0,
               last_real):
    n_blocks = e_pad_h // block_e

    def col_map(i):
        return (0, jnp.minimum(blk0 + i, last_real))

    def full(w):
        return pl.BlockSpec(w.shape, lambda i: (0,) * w.ndim)

    r = lambda x: jnp.reshape(x, (1, -1))
    esw01 = jnp.concatenate([l0['sw'], l1['sw']], axis=1)
    esb01 = jnp.concatenate([r(l0['sb']), r(l1['sb'])], axis=1)
    evw01 = jnp.concatenate([l0['vw'], l1['vw']], axis=1)
    evb01 = jnp.concatenate([r(l0['vb']), r(l1['vb'])], axis=1)
    weights = [we['sw'], r(we['sb']), we['vw'], r(we['vb']),
               esw01, esb01, evw01, evb01]
    out_shape = [jax.ShapeDtypeStruct((e_pad_h, 128), jnp.float32),
                 jax.ShapeDtypeStruct((e_pad_h, 32), jnp.float32)]
    return pl.pallas_call(
        _edge_body,
        grid=(n_blocks,),
        in_specs=[pl.BlockSpec((32, block_e), col_map)] +
                 [pl.BlockSpec((1, block_e), col_map)] * 3 +
                 [full(w) for w in weights],
        out_specs=[pl.BlockSpec((block_e, 128), lambda i: (i, 0)),
                   pl.BlockSpec((block_e, 32), lambda i: (i, 0))],
        out_shape=out_shape,
    )(edge_s_t, *edge_v_comps, *weights)


def _sc_scatter(dst2d, p1, p2, zeros, n_acc, dst_off):
    g = p1.shape[0]
    gpt = g // (_NC * _NS)
    nb = gpt // _GB
    pt = n_acc // _NS
    mesh = plsc.VectorSubcoreMesh(core_axis_name="c", subcore_axis_name="s")

    @functools.partial(
        pl.kernel, mesh=mesh,
        compiler_params=pltpu.CompilerParams(use_tc_tiling_on_sc=False),
        out_type=jax.ShapeDtypeStruct((10, n_acc, 32), jnp.float32),
        scratch_types=[
            pltpu.VMEM_SHARED((n_acc, 32), jnp.float32),
            pltpu.VMEM((gpt, _LG), jnp.int32),
            pltpu.VMEM((_GB, _LG, 32), jnp.float32),
            pltpu.SemaphoreType.DMA,
        ],
    )
    def k(dst_hbm, p1_hbm, p2_hbm, z_hbm, out_hbm, acc, idxs, rowb, sem):
        cid = lax.axis_index("c")
        sid = lax.axis_index("s")
        tid = cid * _NS + sid
        row0 = tid * gpt
        my_rows = pl.ds(sid * pt, pt)
        pltpu.sync_copy(dst_hbm.at[pl.ds(dst_off + row0, gpt)], idxs)
        for ci in range(5):
            pltpu.sync_copy(z_hbm.at[my_rows], acc.at[my_rows])
            plsc.subcore_barrier()

            def body(b, _, ci=ci):
                r = row0 + b * _GB
                if ci < 4:
                    pltpu.sync_copy(
                        p1_hbm.at[pl.ds(r, _GB), :, pl.ds(ci * 32, 32)], rowb)
                else:
                    pltpu.sync_copy(p2_hbm.at[pl.ds(r, _GB)], rowb)
                descs = [
                    pltpu.async_copy(rowb.at[j], acc.at[idxs.at[b * _GB + j]],
                                     sem, add=True)
                    for j in range(_GB)
                ]
                for d in descs:
                    d.wait()
                return _

            lax.fori_loop(0, nb, body, 0)
            plsc.subcore_barrier()
            pltpu.sync_copy(acc.at[my_rows],
                            out_hbm.at[ci * 2 + cid, my_rows])

    return k(dst2d, p1, p2, zeros)


def _node_body(ns_ref, nv_ref, pa_ref, pb_ref,
               wvsw, wvsb, wvvw, wvvb,
               n0sw, n0sb, n0vw, n0vb, r0sw, r0sb, r0vw, r0vb,
               n1sw, n1sb, n1vw, n1vb, r1sw, r1sb, r1vw, r1vb,
               o1sw, o1sb, o2sw, o2sb,
               out_ref):
    sel = _sel_gate(4)
    s_t = _ln_t(ns_ref[...])
    v_t = nv_ref[...]
    m = jnp.sum(v_t * v_t, axis=0, keepdims=True) * 0.25
    vn_t = v_t * lax.rsqrt(m + EPS)
    hs = _dgt(s_t, wvsw[...]) + wvsb[...]
    hv = (_dgt(vn_t, wvvw[...]) + wvvb[...]) * (jax.nn.sigmoid(hs[:, :4]) @ sel)

    p = pa_ref[...] + pb_ref[...]
    pv = p[8] + p[9]
    inv = 1.0 / jnp.maximum(pv[:, 24:25], 1.0)
    aggs = (jnp.concatenate([p[0] + p[1], p[2] + p[3]], axis=1) * inv,
            jnp.concatenate([p[4] + p[5], p[6] + p[7]], axis=1) * inv)
    aggv = (pv[:, 0:12] * inv, pv[:, 12:24] * inv)

    layer_w = ((n0sw, n0sb, n0vw, n0vb, r0sw, r0sb, r0vw, r0vb),
               (n1sw, n1sb, n1vw, n1vb, r1sw, r1sb, r1vw, r1vb))
    for l, (nsw, nsb, nvw, nvb, rsw, rsb, rvw, rvb) in enumerate(layer_w):
        nsn = _ln_s(hs)
        nvn = _ln_v(hv, 4)
        xs = nsn + aggs[l]
        xv = nvn + aggv[l]
        h = _silu(xs @ nsw[...] + nsb[...])
        hv2 = (xv @ nvw[...] + nvb[...]) * (jax.nn.sigmoid(h[:, :4]) @ sel)
        hs = h + nsn @ rsw[...] + rsb[...]
        hv = hv2 + nvn @ rvw[...] + rvb[...]

    nsn = _ln_s(hs)
    h1 = nsn @ o1sw[...] + o1sb[...]
    out_ref[...] = h1 @ o2sw[...] + o2sb[...]


def _node_call(node_s_t, node_v_t, partials, params, block_n):
    n = node_s_t.shape[1]
    n_blocks = -(-n // block_n)

    def full(w):
        return pl.BlockSpec(w.shape, lambda i: (0,) * w.ndim)

    r = lambda x: jnp.reshape(x, (1, -1))
    wv = params['W_v']
    wvvw_perm = jnp.reshape(wv['vw'], (4, 3, 12)).transpose(1, 0, 2).reshape(12, 12)
    lw = []
    for lp in params['layers']:
        ng = lp['node_gvp']
        lw += [ng['sw'], r(ng['sb']), ng['vw'], r(ng['vb']),
               lp['res_s_w'], r(lp['res_s_b']), lp['res_v_w'], r(lp['res_v_b'])]
    o1 = params['W_out1']
    o2 = params['W_out2']
    weights = ([wv['sw'], r(wv['sb']), wvvw_perm, r(wv['vb'])] + lw +
               [o1['sw'], r(o1['sb']), o2['sw'], r(o2['sb'])])
    return pl.pallas_call(
        _node_body,
        grid=(n_blocks,),
        in_specs=[pl.BlockSpec((78, block_n), lambda i: (0, i)),
                  pl.BlockSpec((12, block_n), lambda i: (0, i)),
                  pl.BlockSpec((10, block_n, 32), lambda i: (0, i, 0)),
                  pl.BlockSpec((10, block_n, 32), lambda i: (0, i, 0))] +
                 [full(w) for w in weights],
        out_specs=pl.BlockSpec((block_n, 4), lambda i: (i, 0)),
        out_shape=jax.ShapeDtypeStruct((n, 4), jnp.float32),
    )(node_s_t, node_v_t, *partials, *weights)


def kernel(node_s, node_v, edge_index, edge_s, edge_v, params):
    n = node_s.shape[0]
    e = edge_s.shape[0]
    block_e = 3200
    block_n = 2048
    e_half = e // 2
    edges_per_tile_batch = _NC * _NS * _LG * _GB
    granule = int(np.lcm(edges_per_tile_batch, block_e))
    e_pad_h = -(-e_half // granule) * granule
    n_acc = -(-(n + 1) // (_NS * 8)) * (_NS * 8)

    edge_s_t = edge_s.T
    edge_v_comps = [edge_v[:, 0, c].reshape(1, e) for c in range(3)]
    node_s_t = node_s.T
    node_v_t = jnp.transpose(node_v, (2, 1, 0)).reshape(12, n)
    dst = edge_index[1]
    dst2 = jnp.reshape(dst, (2, e_half))
    dst_ph = jnp.full((2, e_pad_h), n_acc - 1,
                      jnp.int32).at[:, :e_half].set(dst2)
    dst2d = jnp.reshape(dst_ph, (2 * e_pad_h // _LG, _LG))
    zeros = jnp.zeros((n_acc, 32), jnp.float32)

    blocks_h = e_half // block_e
    partials = []
    for h in range(2):
        p1, p2 = _edge_call(edge_s_t, edge_v_comps, params['W_e'],
                            params['layers'][0]['edge_gvp'],
                            params['layers'][1]['edge_gvp'], e_pad_h, block_e,
                            h * blocks_h, h * blocks_h + blocks_h - 1)
        p1r = jnp.reshape(p1, (e_pad_h // _LG, _LG, 128))
        p2r = jnp.reshape(p2, (e_pad_h // _LG, _LG, 32))
        partials.append(_sc_scatter(dst2d, p1r, p2r, zeros, n_acc,
                                    h * (e_pad_h // _LG)))
    return _node_call(node_s_t, node_v_t, partials, params, block_n)
